# baseline (speedup 1.0000x reference)
import functools

import jax
import jax.numpy as jnp
from jax import lax
from jax.experimental import pallas as pl
from jax.experimental.pallas import tpu as pltpu
from jax.experimental.pallas import tpu_sc as plsc

B = 8
N = 16384
C = 256
K = 1024
NV = N // 16
KV = K // 16


def _dist_body(ctx_ref, anc_ref, out_ref):
    dx = ctx_ref[0] - anc_ref[0]
    dy = ctx_ref[1] - anc_ref[1]
    dz = ctx_ref[2] - anc_ref[2]
    out_ref[...] = jnp.sqrt((dx * dx + dy * dy) + dz * dz)


_dist_call = pl.pallas_call(
    _dist_body,
    out_shape=jax.ShapeDtypeStruct((B, N), jnp.float32),
)


_mesh = plsc.VectorSubcoreMesh(core_axis_name="c", subcore_axis_name="s")


@functools.partial(
    pl.kernel,
    out_type=(
        jax.ShapeDtypeStruct((B * K,), jnp.int32),
        jax.ShapeDtypeStruct((B * 3 * K,), jnp.float32),
    ),
    mesh=_mesh,
    compiler_params=pltpu.CompilerParams(needs_layout_passes=False),
    scratch_types=(
        pltpu.VMEM((N,), jnp.int32),
        pltpu.VMEM((N,), jnp.int32),
        pltpu.VMEM((N,), jnp.int32),
        pltpu.VMEM((N,), jnp.int32),
        pltpu.VMEM((256,), jnp.int32),
        pltpu.VMEM((256,), jnp.int32),
        pltpu.VMEM((K,), jnp.int32),
        pltpu.VMEM((K,), jnp.int32),
        pltpu.VMEM((N,), jnp.float32),
        pltpu.VMEM((N,), jnp.float32),
        pltpu.VMEM((N,), jnp.float32),
        pltpu.VMEM((K,), jnp.float32),
        pltpu.VMEM((K,), jnp.float32),
        pltpu.VMEM((K,), jnp.float32),
    ),
)
def _select_call(keys_hbm, ctx_hbm, gidx_hbm, ctxo_hbm,
                 kb0, kb1, ib0, ib1, hist, off, outi, gbuf,
                 ctxx, ctxy, ctxz, ox, oy, oz):
    wid = lax.axis_index("s") * 2 + lax.axis_index("c")
    b = wid

    @pl.when(wid < B)
    def _():
        pltpu.sync_copy(keys_hbm.at[pl.ds(b * N, N)], kb0)
        lanes = lax.iota(jnp.int32, 16)

        def hist_pass(src, shift):
            for j in range(16):
                hist[pl.ds(j * 16, 16)] = jnp.zeros((16,), jnp.int32)

            def body(i, carry):
                v = src[pl.ds(i * 16, 16)]
                d = lax.shift_right_logical(v, shift) & 255
                cnt, last = plsc.scan_count(d)
                plsc.addupdate_scatter(hist, [d], cnt, mask=last)
                return carry

            lax.fori_loop(0, NV, body, 0)
            carry = jnp.int32(0)
            for j in range(16):
                h = hist[pl.ds(j * 16, 16)]
                inc = plsc.cumsum(h) + carry
                off[pl.ds(j * 16, 16)] = inc - h
                carry = jnp.max(inc)

        def permute_pass(srck, srci, dstk, dsti, shift, first, final):
            def body(i, carry):
                v = srck[pl.ds(i * 16, 16)]
                if first:
                    val = lanes + i * 16
                else:
                    val = srci[pl.ds(i * 16, 16)]
                d = lax.shift_right_logical(v, shift) & 255
                cnt, last = plsc.scan_count(d)
                base = plsc.load_gather(off, [d])
                dest = base + (cnt - 1)
                if final:
                    m = dest < K
                    destc = jnp.where(m, dest, 0)
                    plsc.store_scatter(outi, [destc], val, mask=m)
                else:
                    plsc.store_scatter(dstk, [dest], v)
                    plsc.store_scatter(dsti, [dest], val)
                plsc.addupdate_scatter(off, [d], cnt, mask=last)
                return carry

            lax.fori_loop(0, NV, body, 0)

        hist_pass(kb0, 0)
        permute_pass(kb0, None, kb1, ib1, 0, True, False)
        hist_pass(kb1, 8)
        permute_pass(kb1, ib1, kb0, ib0, 8, False, False)
        hist_pass(kb0, 16)
        permute_pass(kb0, ib0, kb1, ib1, 16, False, False)
        hist_pass(kb1, 24)
        permute_pass(kb1, ib1, None, None, 24, False, True)

        pltpu.sync_copy(ctx_hbm.at[pl.ds((b * 3 + 0) * N, N)], ctxx)
        pltpu.sync_copy(ctx_hbm.at[pl.ds((b * 3 + 1) * N, N)], ctxy)
        pltpu.sync_copy(ctx_hbm.at[pl.ds((b * 3 + 2) * N, N)], ctxz)

        def gbody(i, carry):
            iv = outi[pl.ds(i * 16, 16)]
            gbuf[pl.ds(i * 16, 16)] = iv + b * N
            sl = pl.ds(i * 16, 16)
            ox[sl] = plsc.load_gather(ctxx, [iv])
            oy[sl] = plsc.load_gather(ctxy, [iv])
            oz[sl] = plsc.load_gather(ctxz, [iv])
            return carry

        lax.fori_loop(0, KV, gbody, 0)
        pltpu.sync_copy(gbuf, gidx_hbm.at[pl.ds(b * K, K)])
        pltpu.sync_copy(ox, ctxo_hbm.at[pl.ds((b * 3 + 0) * K, K)])
        pltpu.sync_copy(oy, ctxo_hbm.at[pl.ds((b * 3 + 1) * K, K)])
        pltpu.sync_copy(oz, ctxo_hbm.at[pl.ds((b * 3 + 2) * K, K)])


_ROWS_PER_W = (B * K) // 32
_CHUNK = 128


@functools.partial(
    pl.kernel,
    out_type=jax.ShapeDtypeStruct((B * K, C), jnp.float32),
    mesh=_mesh,
    compiler_params=pltpu.CompilerParams(needs_layout_passes=False),
    scratch_types=(
        pltpu.VMEM((_CHUNK,), jnp.int32),
        pltpu.VMEM((_CHUNK,), jnp.int32),
        pltpu.VMEM((_ROWS_PER_W, C), jnp.float32),
        pltpu.SemaphoreType.DMA,
    ),
)
def _gather_call(feats_hbm, gidx_hbm, out_hbm, idxv0, idxv1, rows, sem):
    wid = lax.axis_index("s") * 2 + lax.axis_index("c")
    base = wid * _ROWS_PER_W
    pltpu.sync_copy(gidx_hbm.at[pl.ds(base, _CHUNK)], idxv0)
    pltpu.sync_copy(gidx_hbm.at[pl.ds(base + _CHUNK, _CHUNK)], idxv1)
    copies = [
        pltpu.async_copy(feats_hbm.at[idxv],
                         rows.at[pl.ds(c * _CHUNK, _CHUNK)], sem)
        for c, idxv in enumerate((idxv0, idxv1))
    ]
    for cp in copies:
        cp.wait()
    pltpu.sync_copy(rows, out_hbm.at[pl.ds(base, _ROWS_PER_W)])


def kernel(context_feats, context, anchor, k):
    ctx_planes = jnp.transpose(context, (2, 0, 1))
    anc_planes = jnp.transpose(anchor[:, :3], (1, 0))[:, :, None]
    dist = _dist_call(ctx_planes, anc_planes)
    keys = lax.bitcast_convert_type(dist, jnp.int32)

    ctx_t = jnp.transpose(context, (0, 2, 1))
    gidx, ctxo = _select_call(keys.reshape(B * N), ctx_t.reshape(B * 3 * N))

    g = gidx + (jnp.asarray(k, jnp.int32) - K)
    feats = context_feats.reshape(B * N, C)
    cf = _gather_call(feats, g)
    return (cf.reshape(B, K, C),
            jnp.transpose(ctxo.reshape(B, 3, K), (0, 2, 1)))


# speedup vs baseline: 1.3962x; 1.3962x over previous
import functools

import jax
import jax.numpy as jnp
from jax import lax
from jax.experimental import pallas as pl
from jax.experimental.pallas import tpu as pltpu
from jax.experimental.pallas import tpu_sc as plsc

B = 8
N = 16384
C = 256
K = 1024
NV = N // 16
KV = K // 16


def _dist_body(ctx_ref, anc_ref, out_ref):
    dx = ctx_ref[0] - anc_ref[0]
    dy = ctx_ref[1] - anc_ref[1]
    dz = ctx_ref[2] - anc_ref[2]
    dist = jnp.sqrt((dx * dx + dy * dy) + dz * dz)
    out_ref[...] = lax.bitcast_convert_type(dist, jnp.int32)


_dist_call = pl.pallas_call(
    _dist_body,
    out_shape=jax.ShapeDtypeStruct((B, N), jnp.int32),
)


_mesh = plsc.VectorSubcoreMesh(core_axis_name="c", subcore_axis_name="s")


@functools.partial(
    pl.kernel,
    out_type=(
        jax.ShapeDtypeStruct((B * K,), jnp.int32),
        jax.ShapeDtypeStruct((B * 3 * K,), jnp.float32),
    ),
    mesh=_mesh,
    compiler_params=pltpu.CompilerParams(needs_layout_passes=False),
    scratch_types=(
        pltpu.VMEM((N,), jnp.int32),
        pltpu.VMEM((2048,), jnp.int32),
        pltpu.VMEM((256,), jnp.int32),
        pltpu.VMEM((K,), jnp.int32),
        pltpu.VMEM((K,), jnp.int32),
        pltpu.VMEM((K,), jnp.int32),
        pltpu.VMEM((K,), jnp.int32),
        pltpu.VMEM((K,), jnp.int32),
        pltpu.VMEM((16,), jnp.int32),
        pltpu.VMEM((N,), jnp.float32),
        pltpu.VMEM((N,), jnp.float32),
        pltpu.VMEM((N,), jnp.float32),
        pltpu.VMEM((K,), jnp.float32),
        pltpu.VMEM((K,), jnp.float32),
        pltpu.VMEM((K,), jnp.float32),
    ),
)
def _select_call(keys_hbm, ctx_hbm, kv_hbm, gidx_hbm, ctxo_hbm,
                 kb, hist, off, ck, ci, ck2, ci2, gbuf, kvb,
                 ctxx, ctxy, ctxz, ox, oy, oz):
    wid = lax.axis_index("s") * 2 + lax.axis_index("c")
    b = wid

    @pl.when(wid < B)
    def _():
        pltpu.sync_copy(keys_hbm.at[pl.ds(b * N, N)], kb)
        pltpu.sync_copy(kv_hbm, kvb)
        lanes = lax.iota(jnp.int32, 16)
        zeros16 = jnp.zeros((16,), jnp.int32)

        def zero_hist(nbins):
            def zbody(j, carry):
                hist[pl.ds(j * 16, 16)] = zeros16
                return carry
            lax.fori_loop(0, nbins // 16, zbody, 0)

        def hist_pass(shift, dmask, prefix_shift, prefix):
            def body(i, carry):
                v = kb[pl.ds(i * 16, 16)]
                d = lax.shift_right_logical(v, shift) & dmask
                if prefix_shift < 0:
                    cnt, last = plsc.scan_count(d)
                    plsc.addupdate_scatter(hist, [d], cnt, mask=last)
                else:
                    act = lax.shift_right_logical(v, prefix_shift) == prefix
                    cnt, last = plsc.scan_count(d, mask=act)
                    plsc.addupdate_scatter(hist, [d], cnt, mask=last & act)
                return carry
            lax.fori_loop(0, NV, body, 0)

        def find_digit(nbins, t):
            def body(j, carry):
                dacc, basev, run = carry
                h = hist[pl.ds(j * 16, 16)]
                inc = plsc.cumsum(h) + run
                le = inc <= t
                dacc = dacc + jnp.where(le, 1, 0)
                basev = jnp.maximum(basev, jnp.where(le, inc, 0))
                return dacc, basev, jnp.max(inc)
            dacc, basev, _ = lax.fori_loop(
                0, nbins // 16, body, (zeros16, zeros16, jnp.int32(0)))
            return jnp.sum(dacc), jnp.max(basev)

        t = jnp.int32(K - 1)

        zero_hist(2048)
        hist_pass(20, 2047, -1, None)
        d1, base1 = find_digit(2048, t)
        t = t - base1

        zero_hist(1024)
        hist_pass(10, 1023, 20, d1)
        d2, base2 = find_digit(1024, t)
        t = t - base2

        zero_hist(1024)
        p12 = (d1 << 10) | d2
        hist_pass(0, 1023, 10, p12)
        d3, base3 = find_digit(1024, t)
        t = t - base3

        T = (p12 << 10) | d3
        r = t + 1

        def cbody(i, carry):
            offc, eqc = carry
            v = kb[pl.ds(i * 16, 16)]
            lt = v < T
            eq = v == T
            rc = plsc.cumsum(jnp.where(eq, 1, 0)) + eqc
            sel = lt | (eq & (rc <= r))
            pos = plsc.cumsum(jnp.where(sel, 1, 0))
            dest = offc + pos - 1
            plsc.store_scatter(ck, [dest], v, mask=sel)
            plsc.store_scatter(ci, [dest], lanes + i * 16, mask=sel)
            return offc + jnp.max(pos), jnp.max(rc)
        lax.fori_loop(0, NV, cbody, (jnp.int32(0), jnp.int32(0)))

        def small_sort_pass(srck, srci, dstk, dsti, shift, final):
            def zbody(j, carry):
                hist[pl.ds(j * 16, 16)] = zeros16
                return carry
            lax.fori_loop(0, 16, zbody, 0)

            def hbody(i, carry):
                v = srck[pl.ds(i * 16, 16)]
                d = lax.shift_right_logical(v, shift) & 255
                cnt, last = plsc.scan_count(d)
                plsc.addupdate_scatter(hist, [d], cnt, mask=last)
                return carry
            lax.fori_loop(0, KV, hbody, 0)

            run = jnp.int32(0)
            for j in range(16):
                h = hist[pl.ds(j * 16, 16)]
                inc = plsc.cumsum(h) + run
                off[pl.ds(j * 16, 16)] = inc - h
                run = jnp.max(inc)

            def pbody(i, carry):
                v = srck[pl.ds(i * 16, 16)]
                val = srci[pl.ds(i * 16, 16)]
                d = lax.shift_right_logical(v, shift) & 255
                cnt, last = plsc.scan_count(d)
                base = plsc.load_gather(off, [d])
                dest = base + (cnt - 1)
                if not final:
                    plsc.store_scatter(dstk, [dest], v)
                plsc.store_scatter(dsti, [dest], val)
                plsc.addupdate_scatter(off, [d], cnt, mask=last)
                return carry
            lax.fori_loop(0, KV, pbody, 0)

        small_sort_pass(ck, ci, ck2, ci2, 0, False)
        small_sort_pass(ck2, ci2, ck, ci, 8, False)
        small_sort_pass(ck, ci, ck2, ci2, 16, False)
        small_sort_pass(ck2, ci2, None, ci, 24, True)

        pltpu.sync_copy(ctx_hbm.at[pl.ds((0 * B + b) * N, N)], ctxx)
        pltpu.sync_copy(ctx_hbm.at[pl.ds((1 * B + b) * N, N)], ctxy)
        pltpu.sync_copy(ctx_hbm.at[pl.ds((2 * B + b) * N, N)], ctxz)
        kv = kvb[...]

        def gbody(i, carry):
            sl = pl.ds(i * 16, 16)
            iv = ci[sl]
            gbuf[sl] = iv + b * N + kv
            ox[sl] = plsc.load_gather(ctxx, [iv])
            oy[sl] = plsc.load_gather(ctxy, [iv])
            oz[sl] = plsc.load_gather(ctxz, [iv])
            return carry

        lax.fori_loop(0, KV, gbody, 0)
        pltpu.sync_copy(gbuf, gidx_hbm.at[pl.ds(b * K, K)])
        pltpu.sync_copy(ox, ctxo_hbm.at[pl.ds((b * 3 + 0) * K, K)])
        pltpu.sync_copy(oy, ctxo_hbm.at[pl.ds((b * 3 + 1) * K, K)])
        pltpu.sync_copy(oz, ctxo_hbm.at[pl.ds((b * 3 + 2) * K, K)])


_ROWS_PER_W = (B * K) // 32
_CHUNK = 128


@functools.partial(
    pl.kernel,
    out_type=jax.ShapeDtypeStruct((B * K, C), jnp.float32),
    mesh=_mesh,
    compiler_params=pltpu.CompilerParams(needs_layout_passes=False),
    scratch_types=(
        pltpu.VMEM((_CHUNK,), jnp.int32),
        pltpu.VMEM((_CHUNK,), jnp.int32),
        pltpu.VMEM((_ROWS_PER_W, C), jnp.float32),
        pltpu.SemaphoreType.DMA,
    ),
)
def _gather_call(feats_hbm, gidx_hbm, out_hbm, idxv0, idxv1, rows, sem):
    wid = lax.axis_index("s") * 2 + lax.axis_index("c")
    base = wid * _ROWS_PER_W
    pltpu.sync_copy(gidx_hbm.at[pl.ds(base, _CHUNK)], idxv0)
    pltpu.sync_copy(gidx_hbm.at[pl.ds(base + _CHUNK, _CHUNK)], idxv1)
    copies = [
        pltpu.async_copy(feats_hbm.at[idxv],
                         rows.at[pl.ds(c * _CHUNK, _CHUNK)], sem)
        for c, idxv in enumerate((idxv0, idxv1))
    ]
    for cp in copies:
        cp.wait()
    pltpu.sync_copy(rows, out_hbm.at[pl.ds(base, _ROWS_PER_W)])


def kernel(context_feats, context, anchor, k):
    ctx_planes = jnp.transpose(context, (2, 0, 1))
    anc_planes = jnp.transpose(anchor[:, :3], (1, 0))[:, :, None]
    keys = _dist_call(ctx_planes, anc_planes)

    kvec = jnp.full((16,), jnp.asarray(k, jnp.int32) - K, jnp.int32)
    gidx, ctxo = _select_call(keys.reshape(B * N),
                              ctx_planes.reshape(3 * B * N), kvec)

    feats = context_feats.reshape(B * N, C)
    cf = _gather_call(feats, gidx)
    return (cf.reshape(B, K, C),
            jnp.transpose(ctxo.reshape(B, 3, K), (0, 2, 1)))


# speedup vs baseline: 1.4238x; 1.0198x over previous
import functools

import jax
import jax.numpy as jnp
from jax import lax
from jax.experimental import pallas as pl
from jax.experimental.pallas import tpu as pltpu
from jax.experimental.pallas import tpu_sc as plsc

B = 8
N = 16384
C = 256
K = 1024
NV = N // 16
KV = K // 16


def _dist_body(ctx_ref, anc_ref, out_ref):
    dx = ctx_ref[0] - anc_ref[0]
    dy = ctx_ref[1] - anc_ref[1]
    dz = ctx_ref[2] - anc_ref[2]
    dist = jnp.sqrt((dx * dx + dy * dy) + dz * dz)
    out_ref[...] = lax.bitcast_convert_type(dist, jnp.int32)


_dist_call = pl.pallas_call(
    _dist_body,
    out_shape=jax.ShapeDtypeStruct((B, N), jnp.int32),
)


_mesh = plsc.VectorSubcoreMesh(core_axis_name="c", subcore_axis_name="s")


@functools.partial(
    pl.kernel,
    out_type=(
        jax.ShapeDtypeStruct((B * K,), jnp.int32),
        jax.ShapeDtypeStruct((B * 3 * K,), jnp.float32),
    ),
    mesh=_mesh,
    compiler_params=pltpu.CompilerParams(needs_layout_passes=False),
    scratch_types=(
        pltpu.VMEM((N,), jnp.int32),
        pltpu.VMEM((2048,), jnp.int32),
        pltpu.VMEM((2048,), jnp.int32),
        pltpu.VMEM((2048,), jnp.int32),
        pltpu.VMEM((2048,), jnp.int32),
        pltpu.VMEM((256,), jnp.int32),
        pltpu.VMEM((K,), jnp.int32),
        pltpu.VMEM((K,), jnp.int32),
        pltpu.VMEM((K,), jnp.int32),
        pltpu.VMEM((K,), jnp.int32),
        pltpu.VMEM((K,), jnp.int32),
        pltpu.VMEM((16,), jnp.int32),
        pltpu.VMEM((N,), jnp.float32),
        pltpu.VMEM((N,), jnp.float32),
        pltpu.VMEM((N,), jnp.float32),
        pltpu.VMEM((K,), jnp.float32),
        pltpu.VMEM((K,), jnp.float32),
        pltpu.VMEM((K,), jnp.float32),
    ),
)
def _select_call(keys_hbm, ctx_hbm, kv_hbm, gidx_hbm, ctxo_hbm,
                 kb, hist0, hist1, hist2, hist3, off, ck, ci, ck2, ci2,
                 gbuf, kvb, ctxx, ctxy, ctxz, ox, oy, oz):
    wid = lax.axis_index("s") * 2 + lax.axis_index("c")
    b = wid

    @pl.when(wid < B)
    def _():
        pltpu.sync_copy(keys_hbm.at[pl.ds(b * N, N)], kb)
        pltpu.sync_copy(kv_hbm, kvb)
        lanes = lax.iota(jnp.int32, 16)
        zeros16 = jnp.zeros((16,), jnp.int32)

        hists = (hist0, hist1, hist2, hist3)

        def zero_hist(nbins):
            def zbody(j, carry):
                for h in hists:
                    h[pl.ds(j * 16, 16)] = zeros16
                return carry
            lax.fori_loop(0, nbins // 16, zbody, 0)

        def hist_pass(shift, dmask, prefix_shift, prefix):
            def body(i, carry):
                for u, h in enumerate(hists):
                    v = kb[pl.ds((i * 4 + u) * 16, 16)]
                    d = lax.shift_right_logical(v, shift) & dmask
                    if prefix_shift < 0:
                        cnt, last = plsc.scan_count(d)
                        plsc.addupdate_scatter(h, [d], cnt, mask=last)
                    else:
                        act = lax.shift_right_logical(v, prefix_shift) == prefix
                        cnt, last = plsc.scan_count(d, mask=act)
                        plsc.addupdate_scatter(h, [d], cnt, mask=last & act)
                return carry
            lax.fori_loop(0, NV // 4, body, 0)

        def find_digit(nbins, t):
            def body(j, carry):
                dacc, basev, run = carry
                sl = pl.ds(j * 16, 16)
                h = ((hist0[sl] + hist1[sl]) + (hist2[sl] + hist3[sl]))
                inc = plsc.cumsum(h) + run
                le = inc <= t
                dacc = dacc + jnp.where(le, 1, 0)
                basev = jnp.maximum(basev, jnp.where(le, inc, 0))
                return dacc, basev, jnp.max(inc)
            dacc, basev, _ = lax.fori_loop(
                0, nbins // 16, body, (zeros16, zeros16, jnp.int32(0)))
            return jnp.sum(dacc), jnp.max(basev)

        t = jnp.int32(K - 1)

        zero_hist(2048)
        hist_pass(20, 2047, -1, None)
        d1, base1 = find_digit(2048, t)
        t = t - base1

        zero_hist(1024)
        hist_pass(10, 1023, 20, d1)
        d2, base2 = find_digit(1024, t)
        t = t - base2

        zero_hist(1024)
        p12 = (d1 << 10) | d2
        hist_pass(0, 1023, 10, p12)
        d3, base3 = find_digit(1024, t)
        t = t - base3

        T = (p12 << 10) | d3
        r = t + 1

        def cbody(i, carry):
            offv, eqv = carry
            v = kb[pl.ds(i * 16, 16)]
            lt = v < T
            eq = v == T
            rc = plsc.cumsum(jnp.where(eq, 1, 0)) + eqv
            sel = lt | (eq & (rc <= r))
            pos = plsc.cumsum(jnp.where(sel, 1, 0))
            dest = offv + pos - 1
            plsc.store_scatter(ck, [dest], v, mask=sel)
            plsc.store_scatter(ci, [dest], lanes + i * 16, mask=sel)
            return (offv + plsc.all_reduce_population_count(sel),
                    eqv + plsc.all_reduce_population_count(eq))
        lax.fori_loop(0, NV, cbody, (zeros16, zeros16))

        def small_sort_pass(srck, srci, dstk, dsti, shift, final):
            def zbody(j, carry):
                hist0[pl.ds(j * 16, 16)] = zeros16
                return carry
            lax.fori_loop(0, 16, zbody, 0)

            def hbody(i, carry):
                v = srck[pl.ds(i * 16, 16)]
                d = lax.shift_right_logical(v, shift) & 255
                cnt, last = plsc.scan_count(d)
                plsc.addupdate_scatter(hist0, [d], cnt, mask=last)
                return carry
            lax.fori_loop(0, KV, hbody, 0)

            run = jnp.int32(0)
            for j in range(16):
                h = hist0[pl.ds(j * 16, 16)]
                inc = plsc.cumsum(h) + run
                off[pl.ds(j * 16, 16)] = inc - h
                run = jnp.max(inc)

            def pbody(i, carry):
                v = srck[pl.ds(i * 16, 16)]
                val = srci[pl.ds(i * 16, 16)]
                d = lax.shift_right_logical(v, shift) & 255
                cnt, last = plsc.scan_count(d)
                base = plsc.load_gather(off, [d])
                dest = base + (cnt - 1)
                if not final:
                    plsc.store_scatter(dstk, [dest], v)
                plsc.store_scatter(dsti, [dest], val)
                plsc.addupdate_scatter(off, [d], cnt, mask=last)
                return carry
            lax.fori_loop(0, KV, pbody, 0)

        small_sort_pass(ck, ci, ck2, ci2, 0, False)
        small_sort_pass(ck2, ci2, ck, ci, 8, False)
        small_sort_pass(ck, ci, ck2, ci2, 16, False)
        small_sort_pass(ck2, ci2, None, ci, 24, True)

        pltpu.sync_copy(ctx_hbm.at[pl.ds((0 * B + b) * N, N)], ctxx)
        pltpu.sync_copy(ctx_hbm.at[pl.ds((1 * B + b) * N, N)], ctxy)
        pltpu.sync_copy(ctx_hbm.at[pl.ds((2 * B + b) * N, N)], ctxz)
        kv = kvb[...]

        def gbody(i, carry):
            sl = pl.ds(i * 16, 16)
            iv = ci[sl]
            gbuf[sl] = iv + b * N + kv
            ox[sl] = plsc.load_gather(ctxx, [iv])
            oy[sl] = plsc.load_gather(ctxy, [iv])
            oz[sl] = plsc.load_gather(ctxz, [iv])
            return carry

        lax.fori_loop(0, KV, gbody, 0)
        pltpu.sync_copy(gbuf, gidx_hbm.at[pl.ds(b * K, K)])
        pltpu.sync_copy(ox, ctxo_hbm.at[pl.ds((b * 3 + 0) * K, K)])
        pltpu.sync_copy(oy, ctxo_hbm.at[pl.ds((b * 3 + 1) * K, K)])
        pltpu.sync_copy(oz, ctxo_hbm.at[pl.ds((b * 3 + 2) * K, K)])


_ROWS_PER_W = (B * K) // 32
_CHUNK = 128


@functools.partial(
    pl.kernel,
    out_type=jax.ShapeDtypeStruct((B * K, C), jnp.float32),
    mesh=_mesh,
    compiler_params=pltpu.CompilerParams(needs_layout_passes=False),
    scratch_types=(
        pltpu.VMEM((_CHUNK,), jnp.int32),
        pltpu.VMEM((_CHUNK,), jnp.int32),
        pltpu.VMEM((_ROWS_PER_W, C), jnp.float32),
        pltpu.SemaphoreType.DMA,
    ),
)
def _gather_call(feats_hbm, gidx_hbm, out_hbm, idxv0, idxv1, rows, sem):
    wid = lax.axis_index("s") * 2 + lax.axis_index("c")
    base = wid * _ROWS_PER_W
    pltpu.sync_copy(gidx_hbm.at[pl.ds(base, _CHUNK)], idxv0)
    pltpu.sync_copy(gidx_hbm.at[pl.ds(base + _CHUNK, _CHUNK)], idxv1)
    copies = [
        pltpu.async_copy(feats_hbm.at[idxv],
                         rows.at[pl.ds(c * _CHUNK, _CHUNK)], sem)
        for c, idxv in enumerate((idxv0, idxv1))
    ]
    for cp in copies:
        cp.wait()
    pltpu.sync_copy(rows, out_hbm.at[pl.ds(base, _ROWS_PER_W)])


def kernel(context_feats, context, anchor, k):
    ctx_planes = jnp.transpose(context, (2, 0, 1))
    anc_planes = jnp.transpose(anchor[:, :3], (1, 0))[:, :, None]
    keys = _dist_call(ctx_planes, anc_planes)

    kvec = jnp.full((16,), jnp.asarray(k, jnp.int32) - K, jnp.int32)
    gidx, ctxo = _select_call(keys.reshape(B * N),
                              ctx_planes.reshape(3 * B * N), kvec)

    feats = context_feats.reshape(B * N, C)
    cf = _gather_call(feats, gidx)
    return (cf.reshape(B, K, C),
            jnp.transpose(ctxo.reshape(B, 3, K), (0, 2, 1)))


# speedup vs baseline: 1.9896x; 1.3974x over previous
import functools

import jax
import jax.numpy as jnp
from jax import lax
from jax.experimental import pallas as pl
from jax.experimental.pallas import tpu as pltpu
from jax.experimental.pallas import tpu_sc as plsc

B = 8
N = 16384
C = 256
K = 1024
NV = N // 16
KV = K // 16


def _dist_body(ctx_ref, anc_ref, out_ref):
    dx = ctx_ref[0] - anc_ref[0]
    dy = ctx_ref[1] - anc_ref[1]
    dz = ctx_ref[2] - anc_ref[2]
    dist = jnp.sqrt((dx * dx + dy * dy) + dz * dz)
    out_ref[...] = lax.bitcast_convert_type(dist, jnp.int32)


_dist_call = pl.pallas_call(
    _dist_body,
    out_shape=jax.ShapeDtypeStruct((B, N), jnp.int32),
)


_mesh = plsc.VectorSubcoreMesh(core_axis_name="c", subcore_axis_name="s")


@functools.partial(
    pl.kernel,
    out_type=(
        jax.ShapeDtypeStruct((B * K,), jnp.int32),
        jax.ShapeDtypeStruct((B * 3 * K,), jnp.float32),
    ),
    mesh=_mesh,
    compiler_params=pltpu.CompilerParams(needs_layout_passes=False),
    scratch_types=(
        pltpu.VMEM((N,), jnp.int32),
        pltpu.VMEM((2048,), jnp.int32),
        pltpu.VMEM((2048,), jnp.int32),
        pltpu.VMEM((2048,), jnp.int32),
        pltpu.VMEM((2048,), jnp.int32),
        pltpu.VMEM((256,), jnp.int32),
        pltpu.VMEM((K,), jnp.int32),
        pltpu.VMEM((K,), jnp.int32),
        pltpu.VMEM((N,), jnp.int32),
        pltpu.VMEM((N,), jnp.int32),
        pltpu.VMEM((K,), jnp.int32),
        pltpu.VMEM((16,), jnp.int32),
        pltpu.VMEM((N,), jnp.float32),
        pltpu.VMEM((N,), jnp.float32),
        pltpu.VMEM((N,), jnp.float32),
        pltpu.VMEM((K,), jnp.float32),
        pltpu.VMEM((K,), jnp.float32),
        pltpu.VMEM((K,), jnp.float32),
    ),
)
def _select_call(keys_hbm, ctx_hbm, kv_hbm, gidx_hbm, ctxo_hbm,
                 kb, hist0, hist1, hist2, hist3, off, ck, ci, candk, candi,
                 gbuf, kvb, ctxx, ctxy, ctxz, ox, oy, oz):
    wid = lax.axis_index("s") * 2 + lax.axis_index("c")
    b = wid

    @pl.when(wid < B)
    def _():
        pltpu.sync_copy(keys_hbm.at[pl.ds(b * N, N)], kb)
        pltpu.sync_copy(kv_hbm, kvb)
        lanes = lax.iota(jnp.int32, 16)
        zeros16 = jnp.zeros((16,), jnp.int32)

        hists = (hist0, hist1, hist2, hist3)

        def zero_hist(nbins):
            def zbody(j, carry):
                for h in hists:
                    h[pl.ds(j * 16, 16)] = zeros16
                return carry
            lax.fori_loop(0, nbins // 16, zbody, 0)

        def hist_pass1():
            def body(i, carry):
                for u, h in enumerate(hists):
                    v = kb[pl.ds((i * 4 + u) * 16, 16)]
                    d = lax.shift_right_logical(v, 20)
                    cnt, last = plsc.scan_count(d)
                    plsc.addupdate_scatter(h, [d], cnt, mask=last)
                return carry
            lax.fori_loop(0, NV // 4, body, 0)

        def find_digit(nbins, t):
            def body(j, carry):
                dacc, basev, run = carry
                sl = pl.ds(j * 16, 16)
                h = ((hist0[sl] + hist1[sl]) + (hist2[sl] + hist3[sl]))
                inc = plsc.cumsum(h) + run
                le = inc <= t
                dacc = dacc + jnp.where(le, 1, 0)
                basev = jnp.maximum(basev, jnp.where(le, inc, 0))
                return dacc, basev, jnp.max(inc)
            dacc, basev, _ = lax.fori_loop(
                0, nbins // 16, body, (zeros16, zeros16, jnp.int32(0)))
            return jnp.sum(dacc), jnp.max(basev)

        t = jnp.int32(K - 1)
        zero_hist(2048)
        hist_pass1()
        d1, base1 = find_digit(2048, t)
        t = t - base1

        def cbody(i, carry):
            cv = carry
            v = kb[pl.ds(i * 16, 16)]
            d = lax.shift_right_logical(v, 20)
            win = d < d1
            cand = d == d1
            m = jnp.where(win, 1, 0) + jnp.where(cand, 1 << 16, 0)
            cs = plsc.cumsum(m) + cv
            wpos = (cs & 0xFFFF) - 1
            cpos = lax.shift_right_logical(cs, 16) - 1
            idx = lanes + i * 16
            plsc.store_scatter(ck, [wpos], v, mask=win)
            plsc.store_scatter(ci, [wpos], idx, mask=win)
            plsc.store_scatter(candk, [cpos], v, mask=cand)
            plsc.store_scatter(candi, [cpos], idx, mask=cand)
            pw = plsc.all_reduce_population_count(win)
            pc = plsc.all_reduce_population_count(cand)
            return cv + pw + lax.shift_left(pc, 16)
        cv = lax.fori_loop(0, NV, cbody, zeros16)
        cc = lax.shift_right_logical(jnp.max(cv), 16)
        woff = base1

        def z3body(j, carry):
            sl = pl.ds(j * 16, 16)
            hist1[sl] = zeros16
            hist2[sl] = zeros16
            hist3[sl] = zeros16
            return carry
        lax.fori_loop(0, 64, z3body, 0)

        def refine(shift, t, cc, woff):
            def zbody(j, carry):
                hist0[pl.ds(j * 16, 16)] = zeros16
                return carry
            lax.fori_loop(0, 64, zbody, 0)

            nvec = lax.shift_right_logical(cc + 15, 4)

            def hbody(i, carry):
                act = (lanes + i * 16) < cc
                v = candk[pl.ds(i * 16, 16)]
                d = lax.shift_right_logical(v, shift) & 1023
                cnt, last = plsc.scan_count(d, mask=act)
                plsc.addupdate_scatter(hist0, [d], cnt, mask=last & act)
                return carry
            lax.fori_loop(0, nvec, hbody, 0)

            dd, bb = find_digit(1024, t)
            t = t - bb

            def rbody(i, carry):
                cv2 = carry
                gl = lanes + i * 16
                act = gl < cc
                v = candk[pl.ds(i * 16, 16)]
                idx = candi[pl.ds(i * 16, 16)]
                d = lax.shift_right_logical(v, shift) & 1023
                win = act & (d < dd)
                keep = act & (d == dd)
                m = jnp.where(win, 1, 0) + jnp.where(keep, 1 << 16, 0)
                cs = plsc.cumsum(m) + cv2
                wpos = (cs & 0xFFFF) - 1 + woff
                kpos = lax.shift_right_logical(cs, 16) - 1
                plsc.store_scatter(ck, [wpos], v, mask=win)
                plsc.store_scatter(ci, [wpos], idx, mask=win)
                plsc.store_scatter(candk, [kpos], v, mask=keep)
                plsc.store_scatter(candi, [kpos], idx, mask=keep)
                pw = plsc.all_reduce_population_count(win)
                pk = plsc.all_reduce_population_count(keep)
                return cv2 + pw + lax.shift_left(pk, 16)
            cv2 = lax.fori_loop(0, nvec, rbody, zeros16)
            tot = jnp.max(cv2)
            return t, lax.shift_right_logical(tot, 16), woff + (tot & 0xFFFF)

        t, cc, woff = refine(10, t, cc, woff)
        t, cc, woff = refine(0, t, cc, woff)

        r = t + 1

        def abody(i, carry):
            gl = lanes + i * 16
            msk = gl < r
            sl = pl.ds(i * 16, 16)
            plsc.store_scatter(ck, [woff + gl], candk[sl], mask=msk)
            plsc.store_scatter(ci, [woff + gl], candi[sl], mask=msk)
            return carry
        lax.fori_loop(0, lax.shift_right_logical(r + 15, 4), abody, 0)

        def small_sort_pass(srck, srci, dstk, dsti, shift, final):
            def zbody(j, carry):
                hist0[pl.ds(j * 16, 16)] = zeros16
                return carry
            lax.fori_loop(0, 16, zbody, 0)

            def hbody(i, carry):
                v = srck[pl.ds(i * 16, 16)]
                d = lax.shift_right_logical(v, shift) & 255
                cnt, last = plsc.scan_count(d)
                plsc.addupdate_scatter(hist0, [d], cnt, mask=last)
                return carry
            lax.fori_loop(0, KV, hbody, 0)

            run = jnp.int32(0)
            for j in range(16):
                h = hist0[pl.ds(j * 16, 16)]
                inc = plsc.cumsum(h) + run
                off[pl.ds(j * 16, 16)] = inc - h
                run = jnp.max(inc)

            def pbody(i, carry):
                v = srck[pl.ds(i * 16, 16)]
                val = srci[pl.ds(i * 16, 16)]
                d = lax.shift_right_logical(v, shift) & 255
                cnt, last = plsc.scan_count(d)
                base = plsc.load_gather(off, [d])
                dest = base + (cnt - 1)
                if not final:
                    plsc.store_scatter(dstk, [dest], v)
                plsc.store_scatter(dsti, [dest], val)
                plsc.addupdate_scatter(off, [d], cnt, mask=last)
                return carry
            lax.fori_loop(0, KV, pbody, 0)

        small_sort_pass(ck, ci, candk, candi, 0, False)
        small_sort_pass(candk, candi, ck, ci, 8, False)
        small_sort_pass(ck, ci, candk, candi, 16, False)
        small_sort_pass(candk, candi, None, ci, 24, True)

        pltpu.sync_copy(ctx_hbm.at[pl.ds((0 * B + b) * N, N)], ctxx)
        pltpu.sync_copy(ctx_hbm.at[pl.ds((1 * B + b) * N, N)], ctxy)
        pltpu.sync_copy(ctx_hbm.at[pl.ds((2 * B + b) * N, N)], ctxz)
        kv = kvb[...]

        def gbody(i, carry):
            sl = pl.ds(i * 16, 16)
            iv = ci[sl]
            gbuf[sl] = iv + b * N + kv
            ox[sl] = plsc.load_gather(ctxx, [iv])
            oy[sl] = plsc.load_gather(ctxy, [iv])
            oz[sl] = plsc.load_gather(ctxz, [iv])
            return carry

        lax.fori_loop(0, KV, gbody, 0)
        pltpu.sync_copy(gbuf, gidx_hbm.at[pl.ds(b * K, K)])
        pltpu.sync_copy(ox, ctxo_hbm.at[pl.ds((b * 3 + 0) * K, K)])
        pltpu.sync_copy(oy, ctxo_hbm.at[pl.ds((b * 3 + 1) * K, K)])
        pltpu.sync_copy(oz, ctxo_hbm.at[pl.ds((b * 3 + 2) * K, K)])


_ROWS_PER_W = (B * K) // 32
_CHUNK = 128


@functools.partial(
    pl.kernel,
    out_type=jax.ShapeDtypeStruct((B * K, C), jnp.float32),
    mesh=_mesh,
    compiler_params=pltpu.CompilerParams(needs_layout_passes=False),
    scratch_types=(
        pltpu.VMEM((_CHUNK,), jnp.int32),
        pltpu.VMEM((_CHUNK,), jnp.int32),
        pltpu.VMEM((_ROWS_PER_W, C), jnp.float32),
        pltpu.SemaphoreType.DMA,
    ),
)
def _gather_call(feats_hbm, gidx_hbm, out_hbm, idxv0, idxv1, rows, sem):
    wid = lax.axis_index("s") * 2 + lax.axis_index("c")
    base = wid * _ROWS_PER_W
    pltpu.sync_copy(gidx_hbm.at[pl.ds(base, _CHUNK)], idxv0)
    pltpu.sync_copy(gidx_hbm.at[pl.ds(base + _CHUNK, _CHUNK)], idxv1)
    copies = [
        pltpu.async_copy(feats_hbm.at[idxv],
                         rows.at[pl.ds(c * _CHUNK, _CHUNK)], sem)
        for c, idxv in enumerate((idxv0, idxv1))
    ]
    for cp in copies:
        cp.wait()
    pltpu.sync_copy(rows, out_hbm.at[pl.ds(base, _ROWS_PER_W)])


def kernel(context_feats, context, anchor, k):
    ctx_planes = jnp.transpose(context, (2, 0, 1))
    anc_planes = jnp.transpose(anchor[:, :3], (1, 0))[:, :, None]
    keys = _dist_call(ctx_planes, anc_planes)

    kvec = jnp.full((16,), jnp.asarray(k, jnp.int32) - K, jnp.int32)
    gidx, ctxo = _select_call(keys.reshape(B * N),
                              ctx_planes.reshape(3 * B * N), kvec)

    feats = context_feats.reshape(B * N, C)
    cf = _gather_call(feats, gidx)
    return (cf.reshape(B, K, C),
            jnp.transpose(ctxo.reshape(B, 3, K), (0, 2, 1)))
